# baseline (speedup 1.0000x reference)
import functools

import jax
import jax.numpy as jnp
from jax import lax
from jax.experimental import pallas as pl
from jax.experimental.pallas import tpu as pltpu
from jax.experimental.pallas import tpu_sc as plsc

T = 2048
D = 2048
FF = 768
E = 8
LANES = 128
BM = 256
P = T * 2 + E * BM
NB = P // BM
NC = 2
NS = 16
NW = NC * NS
_SC_MESH = dict(core_axis_name="c", subcore_axis_name="s",
                num_cores=NC, num_subcores=NS)


def _router_body(x_ref, gw_ref, idx_ref, w1_ref, w2_ref, cnt_ref):
    x = x_ref[...]
    gw = gw_ref[...]
    logits = lax.dot_general(x.astype(jnp.bfloat16), gw.astype(jnp.bfloat16),
                             (((1,), (1,)), ((), ())),
                             preferred_element_type=jnp.float32)
    col = lax.broadcasted_iota(jnp.int32, (T, LANES), 1)
    lm = jnp.where(col < E, logits, jnp.float32(-1e30))
    m = jnp.max(lm, axis=1, keepdims=True)
    ex = jnp.exp(lm - m)
    p = ex / jnp.sum(ex, axis=1, keepdims=True)
    w1 = jnp.max(p, axis=1, keepdims=True)
    e1 = jnp.min(jnp.where((p == w1) & (col < E), col, LANES - 1),
                 axis=1, keepdims=True)
    oh1 = col == e1
    p2 = jnp.where(oh1 | (col >= E), jnp.float32(-1.0), p)
    w2 = jnp.max(p2, axis=1, keepdims=True)
    e2 = jnp.min(jnp.where(p2 == w2, col, LANES - 1), axis=1, keepdims=True)
    oh2 = col == e2
    den = w1 + w2
    w1n = w1 / den
    w2n = w2 / den
    oh = (oh1 | oh2).astype(jnp.float32)
    ri = lax.broadcasted_iota(jnp.int32, (T, T), 0)
    ci = lax.broadcasted_iota(jnp.int32, (T, T), 1)
    tri = (ci < ri).astype(jnp.float32)
    pos = lax.dot_general(tri, oh, (((1,), (0,)), ((), ())),
                          precision=lax.Precision.HIGHEST)
    counts = jnp.sum(oh, axis=0, keepdims=True).astype(jnp.int32)
    pc = ((counts + (BM - 1)) >> 8) << 8
    ii = lax.broadcasted_iota(jnp.int32, (LANES, LANES), 0)
    jj = lax.broadcasted_iota(jnp.int32, (LANES, LANES), 1)
    upper = (ii < jj).astype(jnp.float32)
    seg = lax.dot_general(pc.astype(jnp.float32), upper,
                          (((1,), (0,)), ((), ())),
                          precision=lax.Precision.HIGHEST).astype(jnp.int32)
    rmat = seg + pos.astype(jnp.int32)
    r1 = jnp.sum(jnp.where(oh1, rmat, 0), axis=1, keepdims=True)
    r2 = jnp.sum(jnp.where(oh2, rmat, 0), axis=1, keepdims=True)
    idx_ref[...] = jnp.where(col == 0, r1, jnp.where(col == 1, r2, 0))
    w1_ref[...] = jnp.broadcast_to(w1n, (T, LANES))
    w2_ref[...] = jnp.broadcast_to(w2n, (T, LANES))
    cnt_ref[...] = counts


def _router(x, gwp, interpret=False):
    return pl.pallas_call(
        _router_body,
        out_shape=(
            jax.ShapeDtypeStruct((T, LANES), jnp.int32),
            jax.ShapeDtypeStruct((T, LANES), jnp.float32),
            jax.ShapeDtypeStruct((T, LANES), jnp.float32),
            jax.ShapeDtypeStruct((1, LANES), jnp.int32),
        ),
        interpret=interpret,
    )(x, gwp)


def _mlp_body(be_ref, bv_ref, xs_ref, wrow_ref, wg_ref, wu_ref, wd_ref,
              out_ref):
    b = pl.program_id(0)

    @pl.when(bv_ref[b] != 0)
    def _():
        xb = xs_ref[...].astype(jnp.bfloat16)
        wg = wg_ref[0].astype(jnp.bfloat16)
        wu = wu_ref[0].astype(jnp.bfloat16)
        wd = wd_ref[0].astype(jnp.bfloat16)
        g = lax.dot_general(xb, wg, (((1,), (1,)), ((), ())),
                            preferred_element_type=jnp.float32)
        u = lax.dot_general(xb, wu, (((1,), (1,)), ((), ())),
                            preferred_element_type=jnp.float32)
        h = g * jax.lax.logistic(g) * u
        y = lax.dot_general(h.astype(jnp.bfloat16), wd,
                            (((1,), (1,)), ((), ())),
                            preferred_element_type=jnp.float32)
        out_ref[...] = y * wrow_ref[...][:, 0:1]


def _mlp(xs, wrow, Wg, Wu, Wd, bexpert, bvalid, interpret=False):
    grid_spec = pltpu.PrefetchScalarGridSpec(
        num_scalar_prefetch=2,
        grid=(NB,),
        in_specs=[
            pl.BlockSpec((BM, D), lambda b, be, bv: (b, 0)),
            pl.BlockSpec((BM, LANES), lambda b, be, bv: (b, 0)),
            pl.BlockSpec((1, FF, D), lambda b, be, bv: (be[b], 0, 0)),
            pl.BlockSpec((1, FF, D), lambda b, be, bv: (be[b], 0, 0)),
            pl.BlockSpec((1, D, FF), lambda b, be, bv: (be[b], 0, 0)),
        ],
        out_specs=pl.BlockSpec((BM, D), lambda b, be, bv: (b, 0)),
    )
    return pl.pallas_call(
        _mlp_body,
        grid_spec=grid_spec,
        out_shape=jax.ShapeDtypeStruct((P, D), jnp.float32),
        interpret=interpret,
    )(bexpert, bvalid, xs, wrow, Wg, Wu, Wd)


CH_D = 16


def _dispatch_body(x_hbm, r1_hbm, r2_hbm, w1r_hbm, w2r_hbm, xs_hbm, wrow_hbm,
                   idx1_v, idx2_v, rows_v, w1r_v, w2r_v,
                   in_sem0, in_sem1, out_sem0, out_sem1):
    wid = lax.axis_index("s") * NC + lax.axis_index("c")
    tpw = T // NW
    nch = tpw // CH_D
    in_sems = [in_sem0, in_sem1]
    out_sems = [out_sem0, out_sem1]

    def load(ch, s):
        base = wid * tpw + ch * CH_D
        return [
            pltpu.async_copy(x_hbm.at[pl.ds(base, CH_D)], rows_v.at[s],
                             in_sems[s]),
            pltpu.async_copy(r1_hbm.at[pl.ds(base, CH_D)], idx1_v.at[s],
                             in_sems[s]),
            pltpu.async_copy(r2_hbm.at[pl.ds(base, CH_D)], idx2_v.at[s],
                             in_sems[s]),
            pltpu.async_copy(w1r_hbm.at[pl.ds(base, CH_D)], w1r_v.at[s],
                             in_sems[s]),
            pltpu.async_copy(w2r_hbm.at[pl.ds(base, CH_D)], w2r_v.at[s],
                             in_sems[s]),
        ]

    def flush(s):
        return [
            pltpu.async_copy(rows_v.at[s], xs_hbm.at[idx1_v.at[s]],
                             out_sems[s]),
            pltpu.async_copy(rows_v.at[s], xs_hbm.at[idx2_v.at[s]],
                             out_sems[s]),
            pltpu.async_copy(w1r_v.at[s], wrow_hbm.at[idx1_v.at[s]],
                             out_sems[s]),
            pltpu.async_copy(w2r_v.at[s], wrow_hbm.at[idx2_v.at[s]],
                             out_sems[s]),
        ]

    loads = {0: load(0, 0)}
    flushes = {}
    for ch in range(nch):
        s = ch % 2
        if ch >= 2:
            for c in flushes[ch - 2]:
                c.wait()
        if ch + 1 < nch:
            loads[ch + 1] = load(ch + 1, 1 - s)
        for c in loads[ch]:
            c.wait()
        flushes[ch] = flush(s)
    for c in flushes[nch - 2]:
        c.wait()
    for c in flushes[nch - 1]:
        c.wait()


def _dispatch(x, r1, r2, w1rep, w2rep):
    return pl.kernel(
        _dispatch_body,
        out_type=(
            jax.ShapeDtypeStruct((P, D), jnp.float32),
            jax.ShapeDtypeStruct((P, LANES), jnp.float32),
        ),
        mesh=plsc.VectorSubcoreMesh(**_SC_MESH),
        scratch_types=[
            pltpu.VMEM((2, CH_D), jnp.int32),
            pltpu.VMEM((2, CH_D), jnp.int32),
            pltpu.VMEM((2, CH_D, D), jnp.float32),
            pltpu.VMEM((2, CH_D, LANES), jnp.float32),
            pltpu.VMEM((2, CH_D, LANES), jnp.float32),
            pltpu.SemaphoreType.DMA,
            pltpu.SemaphoreType.DMA,
            pltpu.SemaphoreType.DMA,
            pltpu.SemaphoreType.DMA,
        ],
    )(x, r1, r2, w1rep, w2rep)


CH_C = 8


def _combine_body(ys_hbm, r1_hbm, r2_hbm, out_hbm,
                  idx1_v, idx2_v, buf1_v, buf2_v,
                  g_sem0, g_sem1, s_sem0, s_sem1):
    wid = lax.axis_index("s") * NC + lax.axis_index("c")
    tpw = T // NW
    nch = tpw // CH_C
    g_sems = [g_sem0, g_sem1]
    s_sems = [s_sem0, s_sem1]

    def gather(ch, s):
        base = wid * tpw + ch * CH_C
        pltpu.sync_copy(r1_hbm.at[pl.ds(base, CH_C)], idx1_v.at[s])
        pltpu.sync_copy(r2_hbm.at[pl.ds(base, CH_C)], idx2_v.at[s])
        return [
            pltpu.async_copy(ys_hbm.at[idx1_v.at[s]], buf1_v.at[s],
                             g_sems[s]),
            pltpu.async_copy(ys_hbm.at[idx2_v.at[s]], buf2_v.at[s],
                             g_sems[s]),
        ]

    gathers = {0: gather(0, 0)}
    stores = {}
    for ch in range(nch):
        s = ch % 2
        if ch + 1 < nch:
            if ch >= 1:
                stores[ch - 1].wait()
            gathers[ch + 1] = gather(ch + 1, 1 - s)
        for c in gathers[ch]:
            c.wait()

        def row_body(i, _, s=s):
            for j in range(D // 16):
                a = buf1_v[s, i, pl.ds(j * 16, 16)]
                b = buf2_v[s, i, pl.ds(j * 16, 16)]
                buf1_v[s, i, pl.ds(j * 16, 16)] = a + b
            return 0

        lax.fori_loop(0, CH_C, row_body, 0)
        base = wid * tpw + ch * CH_C
        stores[ch] = pltpu.async_copy(buf1_v.at[s],
                                      out_hbm.at[pl.ds(base, CH_C)], s_sems[s])
    stores[nch - 2].wait()
    stores[nch - 1].wait()


def _combine(ys, r1, r2):
    return pl.kernel(
        _combine_body,
        out_type=jax.ShapeDtypeStruct((T, D), jnp.float32),
        mesh=plsc.VectorSubcoreMesh(**_SC_MESH),
        scratch_types=[
            pltpu.VMEM((2, CH_C), jnp.int32),
            pltpu.VMEM((2, CH_C), jnp.int32),
            pltpu.VMEM((2, CH_C, D), jnp.float32),
            pltpu.VMEM((2, CH_C, D), jnp.float32),
            pltpu.SemaphoreType.DMA,
            pltpu.SemaphoreType.DMA,
            pltpu.SemaphoreType.DMA,
            pltpu.SemaphoreType.DMA,
        ],
    )(ys, r1, r2)


def _block_map(counts8):
    pc = ((counts8 + (BM - 1)) // BM) * BM
    ends = jnp.cumsum(pc) // BM
    b = jnp.arange(NB, dtype=jnp.int32)
    bexpert = jnp.minimum(
        jnp.sum(ends[None, :] <= b[:, None], axis=1).astype(jnp.int32), E - 1)
    bvalid = (b < ends[-1]).astype(jnp.int32)
    return bexpert, bvalid


def kernel(hidden_states, gate_w, Wg, Wu, Wd):
    bsz, seq, d = hidden_states.shape
    x = hidden_states.reshape(-1, d)
    gwp = jnp.zeros((LANES, D), jnp.float32).at[:E].set(gate_w)
    idx, w1rep, w2rep, cnt = _router(x, gwp)
    r1 = idx[:, 0]
    r2 = idx[:, 1]
    bexpert, bvalid = _block_map(cnt[0, :E])
    xs, wrow = _dispatch(x, r1, r2, w1rep, w2rep)
    ys = _mlp(xs, wrow, Wg, Wu, Wd, bexpert, bvalid)
    out = _combine(ys, r1, r2)
    return out.reshape(bsz, seq, d)


# speedup vs baseline: 1.0062x; 1.0062x over previous
import functools

import jax
import jax.numpy as jnp
from jax import lax
from jax.experimental import pallas as pl
from jax.experimental.pallas import tpu as pltpu
from jax.experimental.pallas import tpu_sc as plsc

T = 2048
D = 2048
FF = 768
E = 8
LANES = 128
BM = 256
P = T * 2 + E * BM
NB = P // BM
NC = 2
NS = 16
NW = NC * NS
_SC_MESH = dict(core_axis_name="c", subcore_axis_name="s",
                num_cores=NC, num_subcores=NS)


def _router_body(x_ref, gw_ref, idx_ref, w1_ref, w2_ref, cnt_ref):
    x = x_ref[...]
    gw = gw_ref[...]
    logits = lax.dot_general(x.astype(jnp.bfloat16), gw.astype(jnp.bfloat16),
                             (((1,), (1,)), ((), ())),
                             preferred_element_type=jnp.float32)
    col = lax.broadcasted_iota(jnp.int32, (T, LANES), 1)
    lm = jnp.where(col < E, logits, jnp.float32(-1e30))
    m = jnp.max(lm, axis=1, keepdims=True)
    ex = jnp.exp(lm - m)
    p = ex / jnp.sum(ex, axis=1, keepdims=True)
    w1 = jnp.max(p, axis=1, keepdims=True)
    e1 = jnp.min(jnp.where((p == w1) & (col < E), col, LANES - 1),
                 axis=1, keepdims=True)
    oh1 = col == e1
    p2 = jnp.where(oh1 | (col >= E), jnp.float32(-1.0), p)
    w2 = jnp.max(p2, axis=1, keepdims=True)
    e2 = jnp.min(jnp.where(p2 == w2, col, LANES - 1), axis=1, keepdims=True)
    oh2 = col == e2
    den = w1 + w2
    w1n = w1 / den
    w2n = w2 / den
    oh = (oh1 | oh2).astype(jnp.float32)
    ri = lax.broadcasted_iota(jnp.int32, (T, T), 0)
    ci = lax.broadcasted_iota(jnp.int32, (T, T), 1)
    tri = (ci < ri).astype(jnp.float32)
    pos = lax.dot_general(tri, oh, (((1,), (0,)), ((), ())),
                          precision=lax.Precision.HIGHEST)
    counts = jnp.sum(oh, axis=0, keepdims=True).astype(jnp.int32)
    pc = ((counts + (BM - 1)) >> 8) << 8
    ii = lax.broadcasted_iota(jnp.int32, (LANES, LANES), 0)
    jj = lax.broadcasted_iota(jnp.int32, (LANES, LANES), 1)
    upper = (ii < jj).astype(jnp.float32)
    seg = lax.dot_general(pc.astype(jnp.float32), upper,
                          (((1,), (0,)), ((), ())),
                          precision=lax.Precision.HIGHEST).astype(jnp.int32)
    rmat = seg + pos.astype(jnp.int32)
    r1 = jnp.sum(jnp.where(oh1, rmat, 0), axis=1, keepdims=True)
    r2 = jnp.sum(jnp.where(oh2, rmat, 0), axis=1, keepdims=True)
    idx_ref[...] = jnp.where(col == 0, r1, jnp.where(col == 1, r2, 0))
    w1_ref[...] = jnp.broadcast_to(w1n, (T, LANES))
    w2_ref[...] = jnp.broadcast_to(w2n, (T, LANES))
    cnt_ref[...] = counts


def _router(x, gwp, interpret=False):
    return pl.pallas_call(
        _router_body,
        out_shape=(
            jax.ShapeDtypeStruct((T, LANES), jnp.int32),
            jax.ShapeDtypeStruct((T, LANES), jnp.float32),
            jax.ShapeDtypeStruct((T, LANES), jnp.float32),
            jax.ShapeDtypeStruct((1, LANES), jnp.int32),
        ),
        interpret=interpret,
    )(x, gwp)


def _mlp_body(be_ref, bv_ref, xs_ref, wrow_ref, wg_ref, wu_ref, wd_ref,
              out_ref):
    b = pl.program_id(0)

    @pl.when(bv_ref[b] != 0)
    def _():
        xb = xs_ref[...].astype(jnp.bfloat16)
        wg = wg_ref[0].astype(jnp.bfloat16)
        wu = wu_ref[0].astype(jnp.bfloat16)
        wd = wd_ref[0].astype(jnp.bfloat16)
        g = lax.dot_general(xb, wg, (((1,), (1,)), ((), ())),
                            preferred_element_type=jnp.float32)
        u = lax.dot_general(xb, wu, (((1,), (1,)), ((), ())),
                            preferred_element_type=jnp.float32)
        h = g * jax.lax.logistic(g) * u
        y = lax.dot_general(h.astype(jnp.bfloat16), wd,
                            (((1,), (1,)), ((), ())),
                            preferred_element_type=jnp.float32)
        out_ref[...] = y * wrow_ref[...][:, 0:1]


def _mlp(xs, wrow, Wg, Wu, Wd, bexpert, bvalid, interpret=False):
    grid_spec = pltpu.PrefetchScalarGridSpec(
        num_scalar_prefetch=2,
        grid=(NB,),
        in_specs=[
            pl.BlockSpec((BM, D), lambda b, be, bv: (b, 0)),
            pl.BlockSpec((BM, LANES), lambda b, be, bv: (b, 0)),
            pl.BlockSpec((1, FF, D), lambda b, be, bv: (be[b], 0, 0)),
            pl.BlockSpec((1, FF, D), lambda b, be, bv: (be[b], 0, 0)),
            pl.BlockSpec((1, D, FF), lambda b, be, bv: (be[b], 0, 0)),
        ],
        out_specs=pl.BlockSpec((BM, D), lambda b, be, bv: (b, 0)),
    )
    return pl.pallas_call(
        _mlp_body,
        grid_spec=grid_spec,
        out_shape=jax.ShapeDtypeStruct((P, D), jnp.float32),
        interpret=interpret,
    )(bexpert, bvalid, xs, wrow, Wg, Wu, Wd)


CH_D = 16


def _dispatch_body(x_hbm, r1_hbm, r2_hbm, w1r_hbm, w2r_hbm, xs_hbm, wrow_hbm,
                   idx1_a, idx1_b, idx2_a, idx2_b, rows_a, rows_b,
                   w1r_a, w1r_b, w2r_a, w2r_b,
                   in_sem0, in_sem1, out_sem0, out_sem1):
    wid = lax.axis_index("s") * NC + lax.axis_index("c")
    tpw = T // NW
    nch = tpw // CH_D
    idx1 = [idx1_a, idx1_b]
    idx2 = [idx2_a, idx2_b]
    rows = [rows_a, rows_b]
    w1r = [w1r_a, w1r_b]
    w2r = [w2r_a, w2r_b]
    in_sems = [in_sem0, in_sem1]
    out_sems = [out_sem0, out_sem1]

    def load(ch, s):
        base = wid * tpw + ch * CH_D
        return [
            pltpu.async_copy(x_hbm.at[pl.ds(base, CH_D)], rows[s],
                             in_sems[s]),
            pltpu.async_copy(r1_hbm.at[pl.ds(base, CH_D)], idx1[s],
                             in_sems[s]),
            pltpu.async_copy(r2_hbm.at[pl.ds(base, CH_D)], idx2[s],
                             in_sems[s]),
            pltpu.async_copy(w1r_hbm.at[pl.ds(base, CH_D)], w1r[s],
                             in_sems[s]),
            pltpu.async_copy(w2r_hbm.at[pl.ds(base, CH_D)], w2r[s],
                             in_sems[s]),
        ]

    def flush(s):
        return [
            pltpu.async_copy(rows[s], xs_hbm.at[idx1[s]], out_sems[s]),
            pltpu.async_copy(rows[s], xs_hbm.at[idx2[s]], out_sems[s]),
            pltpu.async_copy(w1r[s], wrow_hbm.at[idx1[s]], out_sems[s]),
            pltpu.async_copy(w2r[s], wrow_hbm.at[idx2[s]], out_sems[s]),
        ]

    loads = {0: load(0, 0)}
    flushes = {}
    for ch in range(nch):
        s = ch % 2
        if ch >= 1:
            for c in flushes[ch - 1]:
                c.wait()
        if ch + 1 < nch:
            loads[ch + 1] = load(ch + 1, 1 - s)
        for c in loads[ch]:
            c.wait()
        flushes[ch] = flush(s)
    for c in flushes[nch - 1]:
        c.wait()


def _dispatch(x, r1, r2, w1rep, w2rep):
    return pl.kernel(
        _dispatch_body,
        out_type=(
            jax.ShapeDtypeStruct((P, D), jnp.float32),
            jax.ShapeDtypeStruct((P, LANES), jnp.float32),
        ),
        mesh=plsc.VectorSubcoreMesh(**_SC_MESH),
        scratch_types=[
            pltpu.VMEM((CH_D,), jnp.int32),
            pltpu.VMEM((CH_D,), jnp.int32),
            pltpu.VMEM((CH_D,), jnp.int32),
            pltpu.VMEM((CH_D,), jnp.int32),
            pltpu.VMEM((CH_D, D), jnp.float32),
            pltpu.VMEM((CH_D, D), jnp.float32),
            pltpu.VMEM((CH_D, LANES), jnp.float32),
            pltpu.VMEM((CH_D, LANES), jnp.float32),
            pltpu.VMEM((CH_D, LANES), jnp.float32),
            pltpu.VMEM((CH_D, LANES), jnp.float32),
            pltpu.SemaphoreType.DMA,
            pltpu.SemaphoreType.DMA,
            pltpu.SemaphoreType.DMA,
            pltpu.SemaphoreType.DMA,
        ],
    )(x, r1, r2, w1rep, w2rep)


CH_C = 8


def _combine_body(ys_hbm, r1_hbm, r2_hbm, out_hbm,
                  idx1_a, idx1_b, idx2_a, idx2_b,
                  buf1_a, buf1_b, buf2_a, buf2_b,
                  g_sem0, g_sem1, s_sem0, s_sem1):
    wid = lax.axis_index("s") * NC + lax.axis_index("c")
    tpw = T // NW
    nch = tpw // CH_C
    idx1 = [idx1_a, idx1_b]
    idx2 = [idx2_a, idx2_b]
    buf1 = [buf1_a, buf1_b]
    buf2 = [buf2_a, buf2_b]
    g_sems = [g_sem0, g_sem1]
    s_sems = [s_sem0, s_sem1]

    def gather(ch, s):
        base = wid * tpw + ch * CH_C
        pltpu.sync_copy(r1_hbm.at[pl.ds(base, CH_C)], idx1[s])
        pltpu.sync_copy(r2_hbm.at[pl.ds(base, CH_C)], idx2[s])
        return [
            pltpu.async_copy(ys_hbm.at[idx1[s]], buf1[s], g_sems[s]),
            pltpu.async_copy(ys_hbm.at[idx2[s]], buf2[s], g_sems[s]),
        ]

    gathers = {0: gather(0, 0)}
    stores = {}
    for ch in range(nch):
        s = ch % 2
        if ch + 1 < nch:
            if ch >= 1:
                stores[ch - 1].wait()
            gathers[ch + 1] = gather(ch + 1, 1 - s)
        for c in gathers[ch]:
            c.wait()

        def row_body(i, _, s=s):
            for j in range(D // 16):
                a = buf1[s][i, pl.ds(j * 16, 16)]
                b = buf2[s][i, pl.ds(j * 16, 16)]
                buf1[s][i, pl.ds(j * 16, 16)] = a + b
            return 0

        lax.fori_loop(0, CH_C, row_body, 0)
        base = wid * tpw + ch * CH_C
        stores[ch] = pltpu.async_copy(buf1[s],
                                      out_hbm.at[pl.ds(base, CH_C)], s_sems[s])
    stores[nch - 2].wait()
    stores[nch - 1].wait()


def _combine(ys, r1, r2):
    return pl.kernel(
        _combine_body,
        out_type=jax.ShapeDtypeStruct((T, D), jnp.float32),
        mesh=plsc.VectorSubcoreMesh(**_SC_MESH),
        scratch_types=[
            pltpu.VMEM((CH_C,), jnp.int32),
            pltpu.VMEM((CH_C,), jnp.int32),
            pltpu.VMEM((CH_C,), jnp.int32),
            pltpu.VMEM((CH_C,), jnp.int32),
            pltpu.VMEM((CH_C, D), jnp.float32),
            pltpu.VMEM((CH_C, D), jnp.float32),
            pltpu.VMEM((CH_C, D), jnp.float32),
            pltpu.VMEM((CH_C, D), jnp.float32),
            pltpu.SemaphoreType.DMA,
            pltpu.SemaphoreType.DMA,
            pltpu.SemaphoreType.DMA,
            pltpu.SemaphoreType.DMA,
        ],
    )(ys, r1, r2)


def _block_map(counts8):
    pc = ((counts8 + (BM - 1)) // BM) * BM
    ends = jnp.cumsum(pc) // BM
    b = jnp.arange(NB, dtype=jnp.int32)
    bexpert = jnp.minimum(
        jnp.sum(ends[None, :] <= b[:, None], axis=1).astype(jnp.int32), E - 1)
    bvalid = (b < ends[-1]).astype(jnp.int32)
    return bexpert, bvalid


def kernel(hidden_states, gate_w, Wg, Wu, Wd):
    bsz, seq, d = hidden_states.shape
    x = hidden_states.reshape(-1, d)
    gwp = jnp.zeros((LANES, D), jnp.float32).at[:E].set(gate_w)
    idx, w1rep, w2rep, cnt = _router(x, gwp)
    r1 = idx[:, 0]
    r2 = idx[:, 1]
    bexpert, bvalid = _block_map(cnt[0, :E])
    xs, wrow = _dispatch(x, r1, r2, w1rep, w2rep)
    ys = _mlp(xs, wrow, Wg, Wu, Wd, bexpert, bvalid)
    out = _combine(ys, r1, r2)
    return out.reshape(bsz, seq, d)


# speedup vs baseline: 1.0586x; 1.0520x over previous
import functools

import jax
import jax.numpy as jnp
from jax import lax
from jax.experimental import pallas as pl
from jax.experimental.pallas import tpu as pltpu
from jax.experimental.pallas import tpu_sc as plsc

T = 2048
D = 2048
FF = 768
E = 8
LANES = 128
BM = 256
P = T * 2 + E * BM
NB = P // BM
NC = 2
NS = 16
NW = NC * NS
_SC_MESH = dict(core_axis_name="c", subcore_axis_name="s",
                num_cores=NC, num_subcores=NS)


def _router_body(x_ref, gw_ref, idx_ref, w1_ref, w2_ref, cnt_ref):
    x = x_ref[...]
    gw = gw_ref[...]
    logits = lax.dot_general(x.astype(jnp.bfloat16), gw.astype(jnp.bfloat16),
                             (((1,), (1,)), ((), ())),
                             preferred_element_type=jnp.float32)
    col = lax.broadcasted_iota(jnp.int32, (T, LANES), 1)
    lm = jnp.where(col < E, logits, jnp.float32(-1e30))
    m = jnp.max(lm, axis=1, keepdims=True)
    ex = jnp.exp(lm - m)
    p = ex / jnp.sum(ex, axis=1, keepdims=True)
    w1 = jnp.max(p, axis=1, keepdims=True)
    e1 = jnp.min(jnp.where((p == w1) & (col < E), col, LANES - 1),
                 axis=1, keepdims=True)
    oh1 = col == e1
    p2 = jnp.where(oh1 | (col >= E), jnp.float32(-1.0), p)
    w2 = jnp.max(p2, axis=1, keepdims=True)
    e2 = jnp.min(jnp.where(p2 == w2, col, LANES - 1), axis=1, keepdims=True)
    oh2 = col == e2
    den = w1 + w2
    w1n = w1 / den
    w2n = w2 / den
    oh = (oh1 | oh2).astype(jnp.float32)
    ri = lax.broadcasted_iota(jnp.int32, (T, T), 0)
    ci = lax.broadcasted_iota(jnp.int32, (T, T), 1)
    tri = (ci < ri).astype(jnp.bfloat16)
    pos = lax.dot_general(tri, oh.astype(jnp.bfloat16),
                          (((1,), (0,)), ((), ())),
                          preferred_element_type=jnp.float32)
    counts = jnp.sum(oh, axis=0, keepdims=True).astype(jnp.int32)
    pc = ((counts + (BM - 1)) >> 8) << 8
    ii = lax.broadcasted_iota(jnp.int32, (LANES, LANES), 0)
    jj = lax.broadcasted_iota(jnp.int32, (LANES, LANES), 1)
    upper = (ii < jj).astype(jnp.float32)
    seg = lax.dot_general(pc.astype(jnp.float32), upper,
                          (((1,), (0,)), ((), ())),
                          precision=lax.Precision.HIGHEST).astype(jnp.int32)
    rmat = seg + pos.astype(jnp.int32)
    r1 = jnp.sum(jnp.where(oh1, rmat, 0), axis=1, keepdims=True)
    r2 = jnp.sum(jnp.where(oh2, rmat, 0), axis=1, keepdims=True)
    idx_ref[...] = jnp.where(col == 0, r1, jnp.where(col == 1, r2, 0))
    w1_ref[...] = jnp.broadcast_to(w1n, (T, LANES))
    w2_ref[...] = jnp.broadcast_to(w2n, (T, LANES))
    cnt_ref[...] = counts


def _router(x, gwp, interpret=False):
    return pl.pallas_call(
        _router_body,
        out_shape=(
            jax.ShapeDtypeStruct((T, LANES), jnp.int32),
            jax.ShapeDtypeStruct((T, LANES), jnp.float32),
            jax.ShapeDtypeStruct((T, LANES), jnp.float32),
            jax.ShapeDtypeStruct((1, LANES), jnp.int32),
        ),
        interpret=interpret,
    )(x, gwp)


def _mlp_body(be_ref, bv_ref, xs_ref, wrow_ref, wg_ref, wu_ref, wd_ref,
              out_ref):
    b = pl.program_id(0)

    @pl.when(bv_ref[b] != 0)
    def _():
        xb = xs_ref[...].astype(jnp.bfloat16)
        wg = wg_ref[0].astype(jnp.bfloat16)
        wu = wu_ref[0].astype(jnp.bfloat16)
        wd = wd_ref[0].astype(jnp.bfloat16)
        g = lax.dot_general(xb, wg, (((1,), (1,)), ((), ())),
                            preferred_element_type=jnp.float32)
        u = lax.dot_general(xb, wu, (((1,), (1,)), ((), ())),
                            preferred_element_type=jnp.float32)
        h = g * jax.lax.logistic(g) * u
        y = lax.dot_general(h.astype(jnp.bfloat16), wd,
                            (((1,), (1,)), ((), ())),
                            preferred_element_type=jnp.float32)
        out_ref[...] = y * wrow_ref[...][:, 0:1]


def _mlp(xs, wrow, Wg, Wu, Wd, bexpert, bvalid, interpret=False):
    grid_spec = pltpu.PrefetchScalarGridSpec(
        num_scalar_prefetch=2,
        grid=(NB,),
        in_specs=[
            pl.BlockSpec((BM, D), lambda b, be, bv: (b, 0)),
            pl.BlockSpec((BM, LANES), lambda b, be, bv: (b, 0)),
            pl.BlockSpec((1, FF, D), lambda b, be, bv: (be[b], 0, 0)),
            pl.BlockSpec((1, FF, D), lambda b, be, bv: (be[b], 0, 0)),
            pl.BlockSpec((1, D, FF), lambda b, be, bv: (be[b], 0, 0)),
        ],
        out_specs=pl.BlockSpec((BM, D), lambda b, be, bv: (b, 0)),
    )
    return pl.pallas_call(
        _mlp_body,
        grid_spec=grid_spec,
        out_shape=jax.ShapeDtypeStruct((P, D), jnp.float32),
        interpret=interpret,
    )(bexpert, bvalid, xs, wrow, Wg, Wu, Wd)


CH_D = 16


def _dispatch_body(x_hbm, r1_hbm, r2_hbm, w1r_hbm, w2r_hbm, xs_hbm, wrow_hbm,
                   idx1_a, idx1_b, idx2_a, idx2_b, rows_a, rows_b,
                   w1r_a, w1r_b, w2r_a, w2r_b,
                   in_sem0, in_sem1, out_sem0, out_sem1):
    wid = lax.axis_index("s") * NC + lax.axis_index("c")
    tpw = T // NW
    nch = tpw // CH_D
    idx1 = [idx1_a, idx1_b]
    idx2 = [idx2_a, idx2_b]
    rows = [rows_a, rows_b]
    w1r = [w1r_a, w1r_b]
    w2r = [w2r_a, w2r_b]
    in_sems = [in_sem0, in_sem1]
    out_sems = [out_sem0, out_sem1]

    def load(ch, s):
        base = wid * tpw + ch * CH_D
        return [
            pltpu.async_copy(x_hbm.at[pl.ds(base, CH_D)], rows[s],
                             in_sems[s]),
            pltpu.async_copy(r1_hbm.at[pl.ds(base, CH_D)], idx1[s],
                             in_sems[s]),
            pltpu.async_copy(r2_hbm.at[pl.ds(base, CH_D)], idx2[s],
                             in_sems[s]),
            pltpu.async_copy(w1r_hbm.at[pl.ds(base, CH_D)], w1r[s],
                             in_sems[s]),
            pltpu.async_copy(w2r_hbm.at[pl.ds(base, CH_D)], w2r[s],
                             in_sems[s]),
        ]

    def flush(s):
        return [
            pltpu.async_copy(rows[s], xs_hbm.at[idx1[s]], out_sems[s]),
            pltpu.async_copy(rows[s], xs_hbm.at[idx2[s]], out_sems[s]),
            pltpu.async_copy(w1r[s], wrow_hbm.at[idx1[s]], out_sems[s]),
            pltpu.async_copy(w2r[s], wrow_hbm.at[idx2[s]], out_sems[s]),
        ]

    loads = {0: load(0, 0)}
    flushes = {}
    for ch in range(nch):
        s = ch % 2
        if ch >= 1:
            for c in flushes[ch - 1]:
                c.wait()
        if ch + 1 < nch:
            loads[ch + 1] = load(ch + 1, 1 - s)
        for c in loads[ch]:
            c.wait()
        flushes[ch] = flush(s)
    for c in flushes[nch - 1]:
        c.wait()


def _dispatch(x, r1, r2, w1rep, w2rep):
    return pl.kernel(
        _dispatch_body,
        out_type=(
            jax.ShapeDtypeStruct((P, D), jnp.float32),
            jax.ShapeDtypeStruct((P, LANES), jnp.float32),
        ),
        mesh=plsc.VectorSubcoreMesh(**_SC_MESH),
        scratch_types=[
            pltpu.VMEM((CH_D,), jnp.int32),
            pltpu.VMEM((CH_D,), jnp.int32),
            pltpu.VMEM((CH_D,), jnp.int32),
            pltpu.VMEM((CH_D,), jnp.int32),
            pltpu.VMEM((CH_D, D), jnp.float32),
            pltpu.VMEM((CH_D, D), jnp.float32),
            pltpu.VMEM((CH_D, LANES), jnp.float32),
            pltpu.VMEM((CH_D, LANES), jnp.float32),
            pltpu.VMEM((CH_D, LANES), jnp.float32),
            pltpu.VMEM((CH_D, LANES), jnp.float32),
            pltpu.SemaphoreType.DMA,
            pltpu.SemaphoreType.DMA,
            pltpu.SemaphoreType.DMA,
            pltpu.SemaphoreType.DMA,
        ],
    )(x, r1, r2, w1rep, w2rep)


CH_C = 8


def _combine_body(ys_hbm, r1_hbm, r2_hbm, out_hbm,
                  idx1_a, idx1_b, idx2_a, idx2_b,
                  buf1_a, buf1_b, buf2_a, buf2_b,
                  g_sem0, g_sem1, s_sem0, s_sem1):
    wid = lax.axis_index("s") * NC + lax.axis_index("c")
    tpw = T // NW
    nch = tpw // CH_C
    idx1 = [idx1_a, idx1_b]
    idx2 = [idx2_a, idx2_b]
    buf1 = [buf1_a, buf1_b]
    buf2 = [buf2_a, buf2_b]
    g_sems = [g_sem0, g_sem1]
    s_sems = [s_sem0, s_sem1]

    def gather(ch, s):
        base = wid * tpw + ch * CH_C
        pltpu.sync_copy(r1_hbm.at[pl.ds(base, CH_C)], idx1[s])
        pltpu.sync_copy(r2_hbm.at[pl.ds(base, CH_C)], idx2[s])
        return [
            pltpu.async_copy(ys_hbm.at[idx1[s]], buf1[s], g_sems[s]),
            pltpu.async_copy(ys_hbm.at[idx2[s]], buf2[s], g_sems[s]),
        ]

    gathers = {0: gather(0, 0)}
    stores = {}
    for ch in range(nch):
        s = ch % 2
        if ch + 1 < nch:
            if ch >= 1:
                stores[ch - 1].wait()
            gathers[ch + 1] = gather(ch + 1, 1 - s)
        for c in gathers[ch]:
            c.wait()

        def row_body(i, _, s=s):
            for j in range(D // 16):
                a = buf1[s][i, pl.ds(j * 16, 16)]
                b = buf2[s][i, pl.ds(j * 16, 16)]
                buf1[s][i, pl.ds(j * 16, 16)] = a + b
            return 0

        lax.fori_loop(0, CH_C, row_body, 0)
        base = wid * tpw + ch * CH_C
        stores[ch] = pltpu.async_copy(buf1[s],
                                      out_hbm.at[pl.ds(base, CH_C)], s_sems[s])
    stores[nch - 2].wait()
    stores[nch - 1].wait()


def _combine(ys, r1, r2):
    return pl.kernel(
        _combine_body,
        out_type=jax.ShapeDtypeStruct((T, D), jnp.float32),
        mesh=plsc.VectorSubcoreMesh(**_SC_MESH),
        scratch_types=[
            pltpu.VMEM((CH_C,), jnp.int32),
            pltpu.VMEM((CH_C,), jnp.int32),
            pltpu.VMEM((CH_C,), jnp.int32),
            pltpu.VMEM((CH_C,), jnp.int32),
            pltpu.VMEM((CH_C, D), jnp.float32),
            pltpu.VMEM((CH_C, D), jnp.float32),
            pltpu.VMEM((CH_C, D), jnp.float32),
            pltpu.VMEM((CH_C, D), jnp.float32),
            pltpu.SemaphoreType.DMA,
            pltpu.SemaphoreType.DMA,
            pltpu.SemaphoreType.DMA,
            pltpu.SemaphoreType.DMA,
        ],
    )(ys, r1, r2)


def _block_map(counts8):
    pc = ((counts8 + (BM - 1)) // BM) * BM
    ends = jnp.cumsum(pc) // BM
    b = jnp.arange(NB, dtype=jnp.int32)
    bexpert = jnp.minimum(
        jnp.sum(ends[None, :] <= b[:, None], axis=1).astype(jnp.int32), E - 1)
    bvalid = (b < ends[-1]).astype(jnp.int32)
    return bexpert, bvalid


def kernel(hidden_states, gate_w, Wg, Wu, Wd):
    bsz, seq, d = hidden_states.shape
    x = hidden_states.reshape(-1, d)
    gwp = jnp.zeros((LANES, D), jnp.float32).at[:E].set(gate_w)
    idx, w1rep, w2rep, cnt = _router(x, gwp)
    r1 = idx[:, 0]
    r2 = idx[:, 1]
    bexpert, bvalid = _block_map(cnt[0, :E])
    xs, wrow = _dispatch(x, r1, r2, w1rep, w2rep)
    ys = _mlp(xs, wrow, Wg, Wu, Wd, bexpert, bvalid)
    out = _combine(ys, r1, r2)
    return out.reshape(bsz, seq, d)
